# Initial kernel scaffold; baseline (speedup 1.0000x reference)
import jax
import jax.numpy as jnp
from jax.experimental import pallas as pl


def kernel(x, edge_index, edge_attr, batch, params):
    raise NotImplementedError("write your pallas kernel here")


# speedup vs baseline: 6.7549x; 6.7549x over previous
import functools

import jax
import jax.numpy as jnp
from jax import lax
from jax.experimental import pallas as pl
from jax.experimental.pallas import tpu as pltpu
from jax.experimental.pallas import tpu_sc as plsc

F32 = jnp.float32
I32 = jnp.int32
CH = 128


def _tables_body(atab, bond, vw1, vb1, vw2, vb2,
                 wmb0, bm0, wmb1, bm1, wmb2, bm2, bigtab, btabs):
    a = atab[...]
    h1 = jnp.maximum(a @ vw1[...] + vb1[...], 0.0)
    vl = h1 @ vw2[...] + vb2[...]
    mx = jnp.max(vl, axis=1, keepdims=True)
    io8 = lax.broadcasted_iota(I32, (16, 8), 1).astype(F32)
    idx = jnp.min(jnp.where(vl == mx, io8, 8.0), axis=1, keepdims=True)
    pv = idx + 1.0
    oh = (io8 == idx).astype(F32)
    gate = jnp.clip(pv, 1.0, 8.0) / 8.0
    rowid = lax.broadcasted_iota(I32, (16, 1), 0).astype(F32)
    noble = jnp.logical_or(rowid == 4.0, rowid == 5.0).astype(F32)
    wflag = 4.0 * noble + 2.0 * (pv <= 2.0).astype(F32) + (pv <= 1.0).astype(F32)
    z184 = jnp.zeros((16, 184), F32)
    z117 = jnp.zeros((16, 117), F32)
    bigtab[...] = jnp.concatenate([a, oh, z184, vl, pv, gate, wflag, z117], axis=1)
    b = bond[...]
    t0 = b @ wmb0[...] + bm0[...]
    t1 = b @ wmb1[...] + bm1[...]
    t2 = b @ wmb2[...] + bm2[...]
    btabs[...] = jnp.concatenate([t0, t1, t2], axis=0)


def _tables(p):
    atab = jnp.pad(p['atom_table'], ((0, 5), (0, 0)))
    bond = jnp.pad(p['bond_table'], ((0, 3), (0, 0)))
    args = [atab, bond, p['vp_w1'], p['vp_b1'].reshape(1, 32),
            p['vp_w2'], p['vp_b2'].reshape(1, 8)]
    for l in range(3):
        args += [p['g%d_wm' % l][256:320], p['g%d_bm' % l].reshape(1, 256)]
    return pl.pallas_call(
        _tables_body,
        out_shape=[jax.ShapeDtypeStruct((16, 384), F32),
                   jax.ShapeDtypeStruct((24, 256), F32)],
    )(*args)


def _nodef_body(x, bigtab, nodef):
    b = x.shape[0]
    at = jnp.clip(x[:, 0:1].astype(I32), 0, 10)
    oh = (at == lax.broadcasted_iota(I32, (b, 16), 1)).astype(F32)
    nodef[...] = oh @ bigtab[...]


def _nodef(x, bigtab, n, bn):
    return pl.pallas_call(
        _nodef_body,
        grid=(n // bn,),
        in_specs=[pl.BlockSpec((bn, x.shape[1]), lambda i: (i, 0)),
                  pl.BlockSpec((16, 384), lambda i: (0, 0))],
        out_specs=pl.BlockSpec((bn, 384), lambda i: (i, 0)),
        out_shape=jax.ShapeDtypeStruct((n, 384), F32),
    )(x, bigtab)


def _edge_prep_body(n, ea0, rowr, gr):
    bt = jnp.clip(ea0[...].astype(I32), 0, 4)
    gr[...] = bt * n + rowr[...]


def _edge_prep(ea0r, rowr, n):
    r, c = ea0r.shape
    return pl.pallas_call(
        functools.partial(_edge_prep_body, n),
        out_shape=jax.ShapeDtypeStruct((r, c), I32),
    )(ea0r, rowr)


def _emit_T(l, P, btabs_ref, T_ref):
    bt = btabs_ref[...]
    for c in range(2):
        for b in range(5):
            T_ref[c, b] = jnp.maximum(
                P[:, c * 128:(c + 1) * 128]
                + bt[8 * l + b:8 * l + b + 1, c * 128:(c + 1) * 128], 0.0)


def _prep_T0_body(h, btabs, wmh, T):
    _emit_T(0, h[...] @ wmh[...], btabs, T)


def _prep_T0(h0, btabs, wmh, n, bn):
    return pl.pallas_call(
        _prep_T0_body,
        grid=(n // bn,),
        in_specs=[pl.BlockSpec((bn, 256), lambda i: (i, 0)),
                  pl.BlockSpec((24, 256), lambda i: (0, 0)),
                  pl.BlockSpec((256, 256), lambda i: (0, 0))],
        out_specs=pl.BlockSpec((2, 5, bn, 128), lambda i: (0, 0, i, 0)),
        out_shape=jax.ShapeDtypeStruct((2, 5, n, 128), F32),
    )(h0, btabs, wmh)


def _update_body(lnext, h, agg, gate, wut, wm0, wm1, bu, btabs, wmhn, hout, tout):
    z = (h[...] @ wut[...] + agg[0] @ wm0[...] + agg[1] @ wm1[...] + bu[...])
    hn = jnp.maximum(z, 0.0) * gate[...]
    hout[...] = hn
    if lnext is not None:
        _emit_T(lnext, hn @ wmhn[...], btabs, tout)


def _update(l, h, agg, gate, btabs, p, n, bn):
    wu = p['g%d_wu' % l]
    args = [h, agg, gate, wu[0:256], wu[256:384], wu[384:512],
            p['g%d_bu' % l].reshape(1, 256), btabs]
    in_specs = [pl.BlockSpec((bn, 256), lambda i: (i, 0)),
                pl.BlockSpec((2, bn, 128), lambda i: (0, i, 0)),
                pl.BlockSpec((bn, 1), lambda i: (i, 0)),
                pl.BlockSpec((256, 256), lambda i: (0, 0)),
                pl.BlockSpec((128, 256), lambda i: (0, 0)),
                pl.BlockSpec((128, 256), lambda i: (0, 0)),
                pl.BlockSpec((1, 256), lambda i: (0, 0)),
                pl.BlockSpec((24, 256), lambda i: (0, 0))]
    out_shape = [jax.ShapeDtypeStruct((n, 256), F32)]
    out_specs = [pl.BlockSpec((bn, 256), lambda i: (i, 0))]
    lnext = l + 1 if l < 2 else None
    if lnext is not None:
        args.append(p['g%d_wm' % lnext][0:256])
        in_specs.append(pl.BlockSpec((256, 256), lambda i: (0, 0)))
        out_shape.append(jax.ShapeDtypeStruct((2, 5, n, 128), F32))
        out_specs.append(pl.BlockSpec((2, 5, bn, 128), lambda i: (0, 0, i, 0)))
        body = functools.partial(_update_body, lnext)
    else:
        body = lambda h_, a_, g_, w1, w2, w3, b_, bt_, ho: _update_body(
            None, h_, a_, g_, w1, w2, w3, b_, bt_, None, ho, None)
    return pl.pallas_call(
        body, grid=(n // bn,), in_specs=in_specs,
        out_specs=out_specs, out_shape=out_shape)(*args)


def _final_body(h3, degp, pvn, cw1, cb1, cw2, cb2, wa, wb, wv1, wv2, bb1,
                hf_o, viol_o, props_o, qa_o, qb_o):
    dp = degp[...]
    deg = dp[0, :, 0:1] + dp[1, :, 0:1]
    pv = pvn[...]
    viol = jnp.maximum(deg - pv, 0.0)
    hf = h3[...] / (1.0 + viol)
    hf_o[...] = hf
    viol_o[...] = viol
    t = jnp.maximum(hf @ cw1[...] + cb1[...], 0.0)
    props_o[...] = t @ cw2[...] + cb2[...]
    qa_o[...] = hf @ wa[...] + pv * wv1[...] + bb1[...]
    qb_o[...] = hf @ wb[...] + pv * wv2[...]


def _final(h3, degp, pvn, p, n, bn):
    w1 = p['btc_w1']
    args = [h3, degp, pvn, p['cp_w1'], p['cp_b1'].reshape(1, 128),
            p['cp_w2'], p['cp_b2'].reshape(1, 32),
            w1[0:256], w1[256:512], w1[512:513], w1[513:514],
            p['btc_b1'].reshape(1, 128)]
    in_specs = [pl.BlockSpec((bn, 256), lambda i: (i, 0)),
                pl.BlockSpec((2, bn, 128), lambda i: (0, i, 0)),
                pl.BlockSpec((bn, 1), lambda i: (i, 0)),
                pl.BlockSpec((256, 128), lambda i: (0, 0)),
                pl.BlockSpec((1, 128), lambda i: (0, 0)),
                pl.BlockSpec((128, 32), lambda i: (0, 0)),
                pl.BlockSpec((1, 32), lambda i: (0, 0)),
                pl.BlockSpec((256, 128), lambda i: (0, 0)),
                pl.BlockSpec((256, 128), lambda i: (0, 0)),
                pl.BlockSpec((1, 128), lambda i: (0, 0)),
                pl.BlockSpec((1, 128), lambda i: (0, 0)),
                pl.BlockSpec((1, 128), lambda i: (0, 0))]
    out_shape = [jax.ShapeDtypeStruct((n, 256), F32),
                 jax.ShapeDtypeStruct((n, 1), F32),
                 jax.ShapeDtypeStruct((n, 32), F32),
                 jax.ShapeDtypeStruct((n, 128), F32),
                 jax.ShapeDtypeStruct((n, 128), F32)]
    out_specs = [pl.BlockSpec((bn, 256), lambda i: (i, 0)),
                 pl.BlockSpec((bn, 1), lambda i: (i, 0)),
                 pl.BlockSpec((bn, 32), lambda i: (i, 0)),
                 pl.BlockSpec((bn, 128), lambda i: (i, 0)),
                 pl.BlockSpec((bn, 128), lambda i: (i, 0))]
    return pl.pallas_call(_final_body, grid=(n // bn,), in_specs=in_specs,
                          out_specs=out_specs, out_shape=out_shape)(*args)


def _bondmlp_body(b1, ow, w2, b2, w3, b3, out):
    be = b1.shape[0]
    t = jnp.maximum(b1[...] @ w2[...] + b2[...], 0.0)
    logits = t @ w3[...] + b3[...]
    o = ow[...]
    noble = ((o & 4) > 0).astype(F32)
    m2 = ((o & 2) > 0).astype(F32)
    m3 = ((o & 1) > 0).astype(F32)
    pen1 = -100.0 * noble - 50.0 * m3
    ci = lax.broadcasted_iota(I32, (be, 4), 1)
    out[...] = (logits + jnp.where(ci >= 1, pen1, 0.0)
                + jnp.where(ci == 2, -50.0 * m2, 0.0))


def _bondmlp(b1, ow, p, e, be):
    args = [b1, ow, p['btc_w2'], p['btc_b2'].reshape(1, 64),
            p['btc_w3'], p['btc_b3'].reshape(1, 4)]
    in_specs = [pl.BlockSpec((be, 128), lambda i: (i, 0)),
                pl.BlockSpec((be, 1), lambda i: (i, 0)),
                pl.BlockSpec((128, 64), lambda i: (0, 0)),
                pl.BlockSpec((1, 64), lambda i: (0, 0)),
                pl.BlockSpec((64, 4), lambda i: (0, 0)),
                pl.BlockSpec((1, 4), lambda i: (0, 0))]
    return pl.pallas_call(
        _bondmlp_body, grid=(e // be,), in_specs=in_specs,
        out_specs=pl.BlockSpec((be, 4), lambda i: (i, 0)),
        out_shape=jax.ShapeDtypeStruct((e, 4), F32))(*args)


def _zero_rows(zbuf, nrow, ncolv, value=0.0):
    def zrow(r, _):
        for k in range(ncolv):
            zbuf[r, pl.ds(16 * k, 16)] = jnp.full((16,), value, F32)
        return 0
    lax.fori_loop(0, nrow, zrow, 0)


def _edge_agg(tflat, g, col, n, npad, e):
    nchunk = e // CH
    per_tile = npad // 16
    mesh = plsc.VectorSubcoreMesh(core_axis_name="c", subcore_axis_name="s")

    @functools.partial(
        pl.kernel,
        out_type=jax.ShapeDtypeStruct((2, npad, 128), F32),
        mesh=mesh,
        scratch_types=[
            pltpu.VMEM((CH,), I32),
            pltpu.VMEM((CH,), I32),
            pltpu.VMEM((1, CH), I32),
            pltpu.VMEM((CH, 128), F32),
            pltpu.VMEM((128, 128), F32),
            pltpu.VMEM_SHARED((npad, 128), F32),
            pltpu.SemaphoreType.DMA,
        ])
    def k(t_hbm, g_hbm, col_hbm, out_hbm, gbuf, g2buf, cbuf, msg, zbuf,
          agg_sh, sem):
        c = lax.axis_index("c")
        s = lax.axis_index("s")
        _zero_rows(zbuf, 128, 8)
        base = s * per_tile
        for q in range(per_tile // 128):
            pltpu.sync_copy(zbuf.at[pl.ds(0, 128)],
                            agg_sh.at[pl.ds(base + 128 * q, 128)])
        plsc.subcore_barrier()
        coff = c * (5 * n)

        def body(i, _):
            j = s + 16 * i

            @pl.when(j < nchunk)
            def _():
                pltpu.sync_copy(g_hbm.at[pl.ds(j * CH, CH)], gbuf)
                for k in range(8):
                    sl = pl.ds(16 * k, 16)
                    g2buf[sl] = gbuf[sl] + coff
                pltpu.sync_copy(col_hbm.at[pl.ds(j * CH, CH)], cbuf.at[0])
                pltpu.async_copy(t_hbm.at[g2buf], msg, sem).wait()
                pltpu.sync_copy(msg, agg_sh.at[cbuf.at[0]], add=True)
            return 0

        lax.fori_loop(0, nchunk // 16 + 1, body, 0)
        plsc.subcore_barrier()
        pltpu.sync_copy(agg_sh.at[pl.ds(base, per_tile)],
                        out_hbm.at[c, pl.ds(base, per_tile)])

    return k(tflat, g, col)


def _degree(col, n, npad, e):
    nchunk = e // CH
    per_tile = npad // 16
    mesh = plsc.VectorSubcoreMesh(core_axis_name="c", subcore_axis_name="s")

    @functools.partial(
        pl.kernel,
        out_type=jax.ShapeDtypeStruct((2, npad, 128), F32),
        mesh=mesh,
        scratch_types=[
            pltpu.VMEM((1, CH), I32),
            pltpu.VMEM((CH, 128), F32),
            pltpu.VMEM((128, 128), F32),
            pltpu.VMEM_SHARED((npad, 128), F32),
        ])
    def k(col_hbm, out_hbm, cbuf, ones, zbuf, deg_sh):
        c = lax.axis_index("c")
        s = lax.axis_index("s")
        _zero_rows(zbuf, 128, 8)
        _zero_rows(ones, CH, 8, value=1.0)
        base = s * per_tile
        for q in range(per_tile // 128):
            pltpu.sync_copy(zbuf.at[pl.ds(0, 128)],
                            deg_sh.at[pl.ds(base + 128 * q, 128)])
        plsc.subcore_barrier()
        w = s * 2 + c

        def body(i, _):
            j = w + 32 * i

            @pl.when(j < nchunk)
            def _():
                pltpu.sync_copy(col_hbm.at[pl.ds(j * CH, CH)], cbuf.at[0])
                pltpu.sync_copy(ones, deg_sh.at[cbuf.at[0]], add=True)
            return 0

        lax.fori_loop(0, nchunk // 32 + 1, body, 0)
        plsc.subcore_barrier()
        pltpu.sync_copy(deg_sh.at[pl.ds(base, per_tile)],
                        out_hbm.at[c, pl.ds(base, per_tile)])

    return k(col)


def _bond_edge(qa, qb, row, col, n, e):
    nchunk = e // CH
    mesh = plsc.VectorSubcoreMesh(core_axis_name="c", subcore_axis_name="s")

    @functools.partial(
        pl.kernel,
        out_type=jax.ShapeDtypeStruct((e, 128), F32),
        mesh=mesh,
        scratch_types=[
            pltpu.VMEM((CH,), I32),
            pltpu.VMEM((CH,), I32),
            pltpu.VMEM((CH, 128), F32),
            pltpu.VMEM((CH, 128), F32),
            pltpu.SemaphoreType.DMA,
            pltpu.SemaphoreType.DMA,
        ])
    def k(qa_hbm, qb_hbm, row_hbm, col_hbm, b1_o,
          rbuf, cbuf, bufA, bufB, semA, semB):
        c = lax.axis_index("c")
        s = lax.axis_index("s")
        w = s * 2 + c

        def body(i, _):
            j = w + 32 * i

            @pl.when(j < nchunk)
            def _():
                pltpu.sync_copy(row_hbm.at[pl.ds(j * CH, CH)], rbuf)
                pltpu.sync_copy(col_hbm.at[pl.ds(j * CH, CH)], cbuf)
                da = pltpu.async_copy(qa_hbm.at[rbuf], bufA, semA)
                db = pltpu.async_copy(qb_hbm.at[cbuf], bufB, semB)
                da.wait()
                db.wait()

                def rbody(r, _):
                    for k in range(8):
                        sl = pl.ds(16 * k, 16)
                        bufA[r, sl] = jnp.maximum(bufA[r, sl] + bufB[r, sl],
                                                  0.0)
                    return 0

                lax.fori_loop(0, CH, rbody, 0)
                pltpu.sync_copy(bufA, b1_o.at[pl.ds(j * CH, CH)])
            return 0

        lax.fori_loop(0, nchunk // 32 + 1, body, 0)

    return k(qa, qb, row, col)


def _owflags(wfl, row, col, n, e):
    nchunk = e // CH
    mesh = plsc.VectorSubcoreMesh(core_axis_name="c", subcore_axis_name="s")

    @functools.partial(
        pl.kernel,
        out_type=jax.ShapeDtypeStruct((e,), I32),
        mesh=mesh,
        scratch_types=[
            pltpu.VMEM((CH,), I32),
            pltpu.VMEM((CH,), I32),
            pltpu.VMEM((CH,), I32),
            pltpu.VMEM((n,), I32),
        ],
        compiler_params=pltpu.CompilerParams(needs_layout_passes=False))
    def k(w_hbm, row_hbm, col_hbm, ow_o, rbuf, cbuf, owb, wtile):
        c = lax.axis_index("c")
        s = lax.axis_index("s")
        w = s * 2 + c
        pltpu.sync_copy(w_hbm, wtile)

        def body(i, _):
            j = w + 32 * i

            @pl.when(j < nchunk)
            def _():
                pltpu.sync_copy(row_hbm.at[pl.ds(j * CH, CH)], rbuf)
                pltpu.sync_copy(col_hbm.at[pl.ds(j * CH, CH)], cbuf)
                for k in range(8):
                    sl = pl.ds(16 * k, 16)
                    wr = plsc.load_gather(wtile, [rbuf[sl]])
                    wc = plsc.load_gather(wtile, [cbuf[sl]])
                    owb[sl] = wr | wc
                pltpu.sync_copy(owb, ow_o.at[pl.ds(j * CH, CH)])
            return 0

        lax.fori_loop(0, nchunk // 32 + 1, body, 0)

    return k(wfl, row, col)


def kernel(x, edge_index, edge_attr, batch, params):
    p = params
    n, e = x.shape[0], edge_index.shape[1]
    bn = 1000
    row = edge_index[0].astype(I32)
    col = edge_index[1].astype(I32)

    bigtab, btabs = _tables(p)
    nodef = _nodef(x, bigtab, n, bn)
    h0 = nodef[:, 0:256]
    vlog = nodef[:, 256:264]
    pvn = nodef[:, 264:265]
    gate = nodef[:, 265:266]
    wfl = nodef[:, 266].astype(I32)

    gr = _edge_prep(edge_attr[:, 0].reshape(e // CH, CH),
                    row.reshape(e // CH, CH), n)
    g = gr.reshape(e)

    npad = ((n + 639) // 640) * 640
    degp = _degree(col, n, npad, e)

    t = _prep_T0(h0, btabs, p['g0_wm'][0:256], n, bn)
    h = h0
    for l in range(3):
        agg = _edge_agg(t.reshape(10 * n, 128), g, col, n, npad, e)
        outs = _update(l, h, agg, gate, btabs, p, n, bn)
        if l < 2:
            h, t = outs
        else:
            h = outs[0]

    hf, viol, props, qa, qb = _final(h, degp, pvn, p, n, bn)
    b1 = _bond_edge(qa, qb, row, col, n, e)
    ow = _owflags(wfl, row, col, n, e)
    blog = _bondmlp(b1, ow.reshape(e, 1), p, e, 8000)
    return (hf, props, vlog, blog, viol.reshape(n))


# speedup vs baseline: 9.2713x; 1.3725x over previous
import functools

import jax
import jax.numpy as jnp
from jax import lax
from jax.experimental import pallas as pl
from jax.experimental.pallas import tpu as pltpu
from jax.experimental.pallas import tpu_sc as plsc

F32 = jnp.float32
I32 = jnp.int32
CH = 128


def _tables_body(atab, bond, vw1, vb1, vw2, vb2,
                 wmb0, bm0, wmb1, bm1, wmb2, bm2, bigtab, btabs):
    a = atab[...]
    h1 = jnp.maximum(a @ vw1[...] + vb1[...], 0.0)
    vl = h1 @ vw2[...] + vb2[...]
    mx = jnp.max(vl, axis=1, keepdims=True)
    io8 = lax.broadcasted_iota(I32, (16, 8), 1).astype(F32)
    idx = jnp.min(jnp.where(vl == mx, io8, 8.0), axis=1, keepdims=True)
    pv = idx + 1.0
    oh = (io8 == idx).astype(F32)
    gate = jnp.clip(pv, 1.0, 8.0) / 8.0
    rowid = lax.broadcasted_iota(I32, (16, 1), 0).astype(F32)
    noble = jnp.logical_or(rowid == 4.0, rowid == 5.0).astype(F32)
    wflag = 4.0 * noble + 2.0 * (pv <= 2.0).astype(F32) + (pv <= 1.0).astype(F32)
    z184 = jnp.zeros((16, 184), F32)
    z117 = jnp.zeros((16, 117), F32)
    bigtab[...] = jnp.concatenate([a, oh, z184, vl, pv, gate, wflag, z117], axis=1)
    b = bond[...]
    t0 = b @ wmb0[...] + bm0[...]
    t1 = b @ wmb1[...] + bm1[...]
    t2 = b @ wmb2[...] + bm2[...]
    btabs[...] = jnp.concatenate([t0, t1, t2], axis=0)


def _tables(p):
    atab = jnp.pad(p['atom_table'], ((0, 5), (0, 0)))
    bond = jnp.pad(p['bond_table'], ((0, 3), (0, 0)))
    args = [atab, bond, p['vp_w1'], p['vp_b1'].reshape(1, 32),
            p['vp_w2'], p['vp_b2'].reshape(1, 8)]
    for l in range(3):
        args += [p['g%d_wm' % l][256:320], p['g%d_bm' % l].reshape(1, 256)]
    return pl.pallas_call(
        _tables_body,
        out_shape=[jax.ShapeDtypeStruct((16, 384), F32),
                   jax.ShapeDtypeStruct((24, 256), F32)],
    )(*args)


def _nodef_body(x, bigtab, nodef):
    b = x.shape[0]
    at = jnp.clip(x[:, 0:1].astype(I32), 0, 10)
    oh = (at == lax.broadcasted_iota(I32, (b, 16), 1)).astype(F32)
    nodef[...] = oh @ bigtab[...]


def _nodef(x, bigtab, n, bn):
    return pl.pallas_call(
        _nodef_body,
        grid=(n // bn,),
        in_specs=[pl.BlockSpec((bn, x.shape[1]), lambda i: (i, 0)),
                  pl.BlockSpec((16, 384), lambda i: (0, 0))],
        out_specs=pl.BlockSpec((bn, 384), lambda i: (i, 0)),
        out_shape=jax.ShapeDtypeStruct((n, 384), F32),
    )(x, bigtab)


def _edge_prep_body(n, ea0, rowr, gr):
    bt = jnp.clip(ea0[...].astype(I32), 0, 4)
    gr[...] = bt * n + rowr[...]


def _edge_prep(ea0r, rowr, n):
    r, c = ea0r.shape
    return pl.pallas_call(
        functools.partial(_edge_prep_body, n),
        out_shape=jax.ShapeDtypeStruct((r, c), I32),
    )(ea0r, rowr)


def _emit_T(l, P, btabs_ref, T_ref):
    bt = btabs_ref[...]
    for c in range(2):
        for b in range(5):
            T_ref[c, b] = jnp.maximum(
                P[:, c * 128:(c + 1) * 128]
                + bt[8 * l + b:8 * l + b + 1, c * 128:(c + 1) * 128], 0.0)


def _prep_T0_body(h, btabs, wmh, T):
    _emit_T(0, h[...] @ wmh[...], btabs, T)


def _prep_T0(h0, btabs, wmh, n, bn):
    return pl.pallas_call(
        _prep_T0_body,
        grid=(n // bn,),
        in_specs=[pl.BlockSpec((bn, 256), lambda i: (i, 0)),
                  pl.BlockSpec((24, 256), lambda i: (0, 0)),
                  pl.BlockSpec((256, 256), lambda i: (0, 0))],
        out_specs=pl.BlockSpec((2, 5, bn, 128), lambda i: (0, 0, i, 0)),
        out_shape=jax.ShapeDtypeStruct((2, 5, n, 128), F32),
    )(h0, btabs, wmh)


def _update_body(lnext, h, agg, gate, wut, wm0, wm1, bu, btabs, wmhn, hout, tout):
    z = (h[...] @ wut[...] + agg[0] @ wm0[...] + agg[1] @ wm1[...] + bu[...])
    hn = jnp.maximum(z, 0.0) * gate[...]
    hout[...] = hn
    if lnext is not None:
        _emit_T(lnext, hn @ wmhn[...], btabs, tout)


def _update(l, h, agg, gate, btabs, p, n, bn):
    wu = p['g%d_wu' % l]
    args = [h, agg, gate, wu[0:256], wu[256:384], wu[384:512],
            p['g%d_bu' % l].reshape(1, 256), btabs]
    in_specs = [pl.BlockSpec((bn, 256), lambda i: (i, 0)),
                pl.BlockSpec((2, bn, 128), lambda i: (0, i, 0)),
                pl.BlockSpec((bn, 1), lambda i: (i, 0)),
                pl.BlockSpec((256, 256), lambda i: (0, 0)),
                pl.BlockSpec((128, 256), lambda i: (0, 0)),
                pl.BlockSpec((128, 256), lambda i: (0, 0)),
                pl.BlockSpec((1, 256), lambda i: (0, 0)),
                pl.BlockSpec((24, 256), lambda i: (0, 0))]
    out_shape = [jax.ShapeDtypeStruct((n, 256), F32)]
    out_specs = [pl.BlockSpec((bn, 256), lambda i: (i, 0))]
    lnext = l + 1 if l < 2 else None
    if lnext is not None:
        args.append(p['g%d_wm' % lnext][0:256])
        in_specs.append(pl.BlockSpec((256, 256), lambda i: (0, 0)))
        out_shape.append(jax.ShapeDtypeStruct((2, 5, n, 128), F32))
        out_specs.append(pl.BlockSpec((2, 5, bn, 128), lambda i: (0, 0, i, 0)))
        body = functools.partial(_update_body, lnext)
    else:
        body = lambda h_, a_, g_, w1, w2, w3, b_, bt_, ho: _update_body(
            None, h_, a_, g_, w1, w2, w3, b_, bt_, None, ho, None)
    return pl.pallas_call(
        body, grid=(n // bn,), in_specs=in_specs,
        out_specs=out_specs, out_shape=out_shape)(*args)


def _final_body(h3, degp, pvn, cw1, cb1, cw2, cb2, wa, wb, wv1, wv2, bb1,
                hf_o, viol_o, props_o, qa_o, qb_o):
    dp = degp[...]
    deg = dp[0, :, 0:1] + dp[1, :, 0:1]
    pv = pvn[...]
    viol = jnp.maximum(deg - pv, 0.0)
    hf = h3[...] / (1.0 + viol)
    hf_o[...] = hf
    viol_o[...] = viol
    t = jnp.maximum(hf @ cw1[...] + cb1[...], 0.0)
    props_o[...] = t @ cw2[...] + cb2[...]
    qa_o[...] = hf @ wa[...] + pv * wv1[...] + bb1[...]
    qb_o[...] = hf @ wb[...] + pv * wv2[...]


def _final(h3, degp, pvn, p, n, bn):
    w1 = p['btc_w1']
    args = [h3, degp, pvn, p['cp_w1'], p['cp_b1'].reshape(1, 128),
            p['cp_w2'], p['cp_b2'].reshape(1, 32),
            w1[0:256], w1[256:512], w1[512:513], w1[513:514],
            p['btc_b1'].reshape(1, 128)]
    in_specs = [pl.BlockSpec((bn, 256), lambda i: (i, 0)),
                pl.BlockSpec((2, bn, 128), lambda i: (0, i, 0)),
                pl.BlockSpec((bn, 1), lambda i: (i, 0)),
                pl.BlockSpec((256, 128), lambda i: (0, 0)),
                pl.BlockSpec((1, 128), lambda i: (0, 0)),
                pl.BlockSpec((128, 32), lambda i: (0, 0)),
                pl.BlockSpec((1, 32), lambda i: (0, 0)),
                pl.BlockSpec((256, 128), lambda i: (0, 0)),
                pl.BlockSpec((256, 128), lambda i: (0, 0)),
                pl.BlockSpec((1, 128), lambda i: (0, 0)),
                pl.BlockSpec((1, 128), lambda i: (0, 0)),
                pl.BlockSpec((1, 128), lambda i: (0, 0))]
    out_shape = [jax.ShapeDtypeStruct((n, 256), F32),
                 jax.ShapeDtypeStruct((n, 1), F32),
                 jax.ShapeDtypeStruct((n, 32), F32),
                 jax.ShapeDtypeStruct((n, 128), F32),
                 jax.ShapeDtypeStruct((n, 128), F32)]
    out_specs = [pl.BlockSpec((bn, 256), lambda i: (i, 0)),
                 pl.BlockSpec((bn, 1), lambda i: (i, 0)),
                 pl.BlockSpec((bn, 32), lambda i: (i, 0)),
                 pl.BlockSpec((bn, 128), lambda i: (i, 0)),
                 pl.BlockSpec((bn, 128), lambda i: (i, 0))]
    return pl.pallas_call(_final_body, grid=(n // bn,), in_specs=in_specs,
                          out_specs=out_specs, out_shape=out_shape)(*args)


def _bondmlp_body(b1, ow, w2, b2, w3, b3, out):
    be = b1.shape[0]
    t = jnp.maximum(b1[...] @ w2[...] + b2[...], 0.0)
    logits = t @ w3[...] + b3[...]
    o = ow[...]
    noble = ((o & 4) > 0).astype(F32)
    m2 = ((o & 2) > 0).astype(F32)
    m3 = ((o & 1) > 0).astype(F32)
    pen1 = -100.0 * noble - 50.0 * m3
    ci = lax.broadcasted_iota(I32, (be, 4), 1)
    out[...] = (logits + jnp.where(ci >= 1, pen1, 0.0)
                + jnp.where(ci == 2, -50.0 * m2, 0.0))


def _bondmlp(b1, ow, p, e, be):
    args = [b1, ow, p['btc_w2'], p['btc_b2'].reshape(1, 64),
            p['btc_w3'], p['btc_b3'].reshape(1, 4)]
    in_specs = [pl.BlockSpec((be, 128), lambda i: (i, 0)),
                pl.BlockSpec((be, 1), lambda i: (i, 0)),
                pl.BlockSpec((128, 64), lambda i: (0, 0)),
                pl.BlockSpec((1, 64), lambda i: (0, 0)),
                pl.BlockSpec((64, 4), lambda i: (0, 0)),
                pl.BlockSpec((1, 4), lambda i: (0, 0))]
    return pl.pallas_call(
        _bondmlp_body, grid=(e // be,), in_specs=in_specs,
        out_specs=pl.BlockSpec((be, 4), lambda i: (i, 0)),
        out_shape=jax.ShapeDtypeStruct((e, 4), F32))(*args)


def _zero_rows(zbuf, nrow, ncolv, value=0.0):
    def zrow(r, _):
        for k in range(ncolv):
            zbuf[r, pl.ds(16 * k, 16)] = jnp.full((16,), value, F32)
        return 0
    lax.fori_loop(0, nrow, zrow, 0)


def _edge_agg(tflat, g, col, n, npad, e):
    nchunk = e // CH
    per_tile = npad // 16
    mesh = plsc.VectorSubcoreMesh(core_axis_name="c", subcore_axis_name="s")

    @functools.partial(
        pl.kernel,
        out_type=jax.ShapeDtypeStruct((2, npad, 128), F32),
        mesh=mesh,
        scratch_types=[
            pltpu.VMEM((2, CH), I32),
            pltpu.VMEM((2, CH), I32),
            pltpu.VMEM((2, CH), I32),
            pltpu.VMEM((2, CH, 128), F32),
            pltpu.VMEM_SHARED((npad, 128), F32),
            pltpu.SemaphoreType.DMA,
            pltpu.SemaphoreType.DMA,
        ])
    def k(t_hbm, g_hbm, col_hbm, out_hbm, gbuf, g2buf, cbuf, msg,
          agg_sh, semA, semB):
        c = lax.axis_index("c")
        s = lax.axis_index("s")
        zbuf = msg.at[0]
        _zero_rows(zbuf, 128, 8)
        base = s * per_tile
        for q in range(per_tile // 128):
            pltpu.sync_copy(zbuf.at[pl.ds(0, 128)],
                            agg_sh.at[pl.ds(base + 128 * q, 128)])
        plsc.subcore_barrier()
        coff = c * (5 * n)
        sems = (semA, semB)

        def prep(i, b):
            j = s + 16 * i
            pltpu.sync_copy(g_hbm.at[pl.ds(j * CH, CH)], gbuf.at[b])
            for kk in range(8):
                sl = pl.ds(16 * kk, 16)
                g2buf[b, sl] = gbuf[b, sl] + coff
            pltpu.sync_copy(col_hbm.at[pl.ds(j * CH, CH)], cbuf.at[b])

        def start(b):
            pltpu.async_copy(t_hbm.at[g2buf.at[b]], msg.at[b], sems[b])

        def wait(b):
            pltpu.make_async_copy(t_hbm.at[g2buf.at[b]], msg.at[b],
                                  sems[b]).wait()

        def scat(b):
            pltpu.sync_copy(msg.at[b], agg_sh.at[cbuf.at[b]], add=True)

        def v(i):
            return s + 16 * i < nchunk

        prep(0, 0)
        start(0)

        def body(ii, _):
            i0 = 2 * ii
            i1 = i0 + 1
            i2 = i0 + 2

            @pl.when(v(i1))
            def _():
                prep(i1, 1)
                start(1)

            @pl.when(v(i0))
            def _():
                wait(0)
                scat(0)

            @pl.when(v(i2))
            def _():
                prep(i2, 0)
                start(0)

            @pl.when(v(i1))
            def _():
                wait(1)
                scat(1)
            return 0

        lax.fori_loop(0, (nchunk // 16 + 2) // 2, body, 0)
        plsc.subcore_barrier()
        pltpu.sync_copy(agg_sh.at[pl.ds(base, per_tile)],
                        out_hbm.at[c, pl.ds(base, per_tile)])

    return k(tflat, g, col)


def _degree(col, n, npad, e):
    nchunk = e // CH
    per_tile = npad // 16
    mesh = plsc.VectorSubcoreMesh(core_axis_name="c", subcore_axis_name="s")

    @functools.partial(
        pl.kernel,
        out_type=jax.ShapeDtypeStruct((2, npad, 128), F32),
        mesh=mesh,
        scratch_types=[
            pltpu.VMEM((1, CH), I32),
            pltpu.VMEM((CH, 128), F32),
            pltpu.VMEM((128, 128), F32),
            pltpu.VMEM_SHARED((npad, 128), F32),
        ])
    def k(col_hbm, out_hbm, cbuf, ones, zbuf, deg_sh):
        c = lax.axis_index("c")
        s = lax.axis_index("s")
        _zero_rows(zbuf, 128, 8)
        _zero_rows(ones, CH, 8, value=1.0)
        base = s * per_tile
        for q in range(per_tile // 128):
            pltpu.sync_copy(zbuf.at[pl.ds(0, 128)],
                            deg_sh.at[pl.ds(base + 128 * q, 128)])
        plsc.subcore_barrier()
        w = s * 2 + c

        def body(i, _):
            j = w + 32 * i

            @pl.when(j < nchunk)
            def _():
                pltpu.sync_copy(col_hbm.at[pl.ds(j * CH, CH)], cbuf.at[0])
                pltpu.sync_copy(ones, deg_sh.at[cbuf.at[0]], add=True)
            return 0

        lax.fori_loop(0, nchunk // 32 + 1, body, 0)
        plsc.subcore_barrier()
        pltpu.sync_copy(deg_sh.at[pl.ds(base, per_tile)],
                        out_hbm.at[c, pl.ds(base, per_tile)])

    return k(col)


def _bond_edge(qa, qb, row, col, n, e):
    nchunk = e // CH
    mesh = plsc.VectorSubcoreMesh(core_axis_name="c", subcore_axis_name="s")

    @functools.partial(
        pl.kernel,
        out_type=jax.ShapeDtypeStruct((e, 128), F32),
        mesh=mesh,
        scratch_types=[
            pltpu.VMEM((2, CH), I32),
            pltpu.VMEM((2, CH), I32),
            pltpu.VMEM((2, CH, 128), F32),
            pltpu.VMEM((2, CH, 128), F32),
            pltpu.SemaphoreType.DMA,
            pltpu.SemaphoreType.DMA,
            pltpu.SemaphoreType.DMA,
            pltpu.SemaphoreType.DMA,
        ])
    def k(qa_hbm, qb_hbm, row_hbm, col_hbm, b1_o,
          rbuf, cbuf, bufA, bufB, semA0, semA1, semB0, semB1):
        c = lax.axis_index("c")
        s = lax.axis_index("s")
        w = s * 2 + c
        semsA = (semA0, semA1)
        semsB = (semB0, semB1)

        def prep(i, b):
            j = w + 32 * i
            pltpu.sync_copy(row_hbm.at[pl.ds(j * CH, CH)], rbuf.at[b])
            pltpu.sync_copy(col_hbm.at[pl.ds(j * CH, CH)], cbuf.at[b])

        def start(b):
            pltpu.async_copy(qa_hbm.at[rbuf.at[b]], bufA.at[b], semsA[b])
            pltpu.async_copy(qb_hbm.at[cbuf.at[b]], bufB.at[b], semsB[b])

        def wait(b):
            pltpu.make_async_copy(qa_hbm.at[rbuf.at[b]], bufA.at[b],
                                  semsA[b]).wait()
            pltpu.make_async_copy(qb_hbm.at[cbuf.at[b]], bufB.at[b],
                                  semsB[b]).wait()

        def compute(i, b):
            j = w + 32 * i

            def rbody(r, _):
                for kk in range(8):
                    sl = pl.ds(16 * kk, 16)
                    bufA[b, r, sl] = jnp.maximum(
                        bufA[b, r, sl] + bufB[b, r, sl], 0.0)
                return 0

            lax.fori_loop(0, CH, rbody, 0)
            pltpu.sync_copy(bufA.at[b], b1_o.at[pl.ds(j * CH, CH)])

        def v(i):
            return w + 32 * i < nchunk

        prep(0, 0)
        start(0)

        def body(ii, _):
            i0 = 2 * ii
            i1 = i0 + 1
            i2 = i0 + 2

            @pl.when(v(i1))
            def _():
                prep(i1, 1)
                start(1)

            @pl.when(v(i0))
            def _():
                wait(0)
                compute(i0, 0)

            @pl.when(v(i2))
            def _():
                prep(i2, 0)
                start(0)

            @pl.when(v(i1))
            def _():
                wait(1)
                compute(i1, 1)
            return 0

        lax.fori_loop(0, (nchunk // 32 + 2) // 2, body, 0)

    return k(qa, qb, row, col)


def _owflags(wfl, row, col, n, e):
    nchunk = e // CH
    mesh = plsc.VectorSubcoreMesh(core_axis_name="c", subcore_axis_name="s")

    @functools.partial(
        pl.kernel,
        out_type=jax.ShapeDtypeStruct((e,), I32),
        mesh=mesh,
        scratch_types=[
            pltpu.VMEM((CH,), I32),
            pltpu.VMEM((CH,), I32),
            pltpu.VMEM((CH,), I32),
            pltpu.VMEM((n,), I32),
        ],
        compiler_params=pltpu.CompilerParams(needs_layout_passes=False))
    def k(w_hbm, row_hbm, col_hbm, ow_o, rbuf, cbuf, owb, wtile):
        c = lax.axis_index("c")
        s = lax.axis_index("s")
        w = s * 2 + c
        pltpu.sync_copy(w_hbm, wtile)

        def body(i, _):
            j = w + 32 * i

            @pl.when(j < nchunk)
            def _():
                pltpu.sync_copy(row_hbm.at[pl.ds(j * CH, CH)], rbuf)
                pltpu.sync_copy(col_hbm.at[pl.ds(j * CH, CH)], cbuf)
                for k in range(8):
                    sl = pl.ds(16 * k, 16)
                    wr = plsc.load_gather(wtile, [rbuf[sl]])
                    wc = plsc.load_gather(wtile, [cbuf[sl]])
                    owb[sl] = wr | wc
                pltpu.sync_copy(owb, ow_o.at[pl.ds(j * CH, CH)])
            return 0

        lax.fori_loop(0, nchunk // 32 + 1, body, 0)

    return k(wfl, row, col)


def kernel(x, edge_index, edge_attr, batch, params):
    p = params
    n, e = x.shape[0], edge_index.shape[1]
    bn = 1000
    row = edge_index[0].astype(I32)
    col = edge_index[1].astype(I32)

    bigtab, btabs = _tables(p)
    nodef = _nodef(x, bigtab, n, bn)
    h0 = nodef[:, 0:256]
    vlog = nodef[:, 256:264]
    pvn = nodef[:, 264:265]
    gate = nodef[:, 265:266]
    wfl = nodef[:, 266].astype(I32)

    gr = _edge_prep(edge_attr[:, 0].reshape(e // CH, CH),
                    row.reshape(e // CH, CH), n)
    g = gr.reshape(e)

    npad = ((n + 639) // 640) * 640
    degp = _degree(col, n, npad, e)

    t = _prep_T0(h0, btabs, p['g0_wm'][0:256], n, bn)
    h = h0
    for l in range(3):
        agg = _edge_agg(t.reshape(10 * n, 128), g, col, n, npad, e)
        outs = _update(l, h, agg, gate, btabs, p, n, bn)
        if l < 2:
            h, t = outs
        else:
            h = outs[0]

    hf, viol, props, qa, qb = _final(h, degp, pvn, p, n, bn)
    b1 = _bond_edge(qa, qb, row, col, n, e)
    ow = _owflags(wfl, row, col, n, e)
    blog = _bondmlp(b1, ow.reshape(e, 1), p, e, 8000)
    return (hf, props, vlog, blog, viol.reshape(n))


# speedup vs baseline: 9.7044x; 1.0467x over previous
import functools

import jax
import jax.numpy as jnp
from jax import lax
from jax.experimental import pallas as pl
from jax.experimental.pallas import tpu as pltpu
from jax.experimental.pallas import tpu_sc as plsc

F32 = jnp.float32
I32 = jnp.int32
CH = 128


def _tables_body(atab, bond, vw1, vb1, vw2, vb2,
                 wmb0, bm0, wmb1, bm1, wmb2, bm2, bigtab, btabs):
    a = atab[...]
    h1 = jnp.maximum(a @ vw1[...] + vb1[...], 0.0)
    vl = h1 @ vw2[...] + vb2[...]
    mx = jnp.max(vl, axis=1, keepdims=True)
    io8 = lax.broadcasted_iota(I32, (16, 8), 1).astype(F32)
    idx = jnp.min(jnp.where(vl == mx, io8, 8.0), axis=1, keepdims=True)
    pv = idx + 1.0
    oh = (io8 == idx).astype(F32)
    gate = jnp.clip(pv, 1.0, 8.0) / 8.0
    rowid = lax.broadcasted_iota(I32, (16, 1), 0).astype(F32)
    noble = jnp.logical_or(rowid == 4.0, rowid == 5.0).astype(F32)
    wflag = 4.0 * noble + 2.0 * (pv <= 2.0).astype(F32) + (pv <= 1.0).astype(F32)
    z184 = jnp.zeros((16, 184), F32)
    z117 = jnp.zeros((16, 117), F32)
    bigtab[...] = jnp.concatenate([a, oh, z184, vl, pv, gate, wflag, z117], axis=1)
    b = bond[...]
    t0 = b @ wmb0[...] + bm0[...]
    t1 = b @ wmb1[...] + bm1[...]
    t2 = b @ wmb2[...] + bm2[...]
    btabs[...] = jnp.concatenate([t0, t1, t2], axis=0)


def _tables(p):
    atab = jnp.pad(p['atom_table'], ((0, 5), (0, 0)))
    bond = jnp.pad(p['bond_table'], ((0, 3), (0, 0)))
    args = [atab, bond, p['vp_w1'], p['vp_b1'].reshape(1, 32),
            p['vp_w2'], p['vp_b2'].reshape(1, 8)]
    for l in range(3):
        args += [p['g%d_wm' % l][256:320], p['g%d_bm' % l].reshape(1, 256)]
    return pl.pallas_call(
        _tables_body,
        out_shape=[jax.ShapeDtypeStruct((16, 384), F32),
                   jax.ShapeDtypeStruct((24, 256), F32)],
    )(*args)


def _nodef_body(x, bigtab, nodef):
    b = x.shape[0]
    at = jnp.clip(x[:, 0:1].astype(I32), 0, 10)
    oh = (at == lax.broadcasted_iota(I32, (b, 16), 1)).astype(F32)
    nodef[...] = oh @ bigtab[...]


def _nodef(x, bigtab, n, bn):
    return pl.pallas_call(
        _nodef_body,
        grid=(n // bn,),
        in_specs=[pl.BlockSpec((bn, x.shape[1]), lambda i: (i, 0)),
                  pl.BlockSpec((16, 384), lambda i: (0, 0))],
        out_specs=pl.BlockSpec((bn, 384), lambda i: (i, 0)),
        out_shape=jax.ShapeDtypeStruct((n, 384), F32),
    )(x, bigtab)


def _edge_prep_body(n, ea0, rowr, gr):
    bt = jnp.clip(ea0[...].astype(I32), 0, 4)
    gr[...] = bt * n + rowr[...]


def _edge_prep(ea0r, rowr, n):
    r, c = ea0r.shape
    return pl.pallas_call(
        functools.partial(_edge_prep_body, n),
        out_shape=jax.ShapeDtypeStruct((r, c), I32),
    )(ea0r, rowr)


def _emit_T(l, P, btabs_ref, T_ref):
    bt = btabs_ref[...]
    for c in range(2):
        for b in range(5):
            T_ref[c, b] = jnp.maximum(
                P[:, c * 128:(c + 1) * 128]
                + bt[8 * l + b:8 * l + b + 1, c * 128:(c + 1) * 128], 0.0)


def _prep_T0_body(h, btabs, wmh, T):
    _emit_T(0, h[...] @ wmh[...], btabs, T)


def _prep_T0(h0, btabs, wmh, n, bn):
    return pl.pallas_call(
        _prep_T0_body,
        grid=(n // bn,),
        in_specs=[pl.BlockSpec((bn, 256), lambda i: (i, 0)),
                  pl.BlockSpec((24, 256), lambda i: (0, 0)),
                  pl.BlockSpec((256, 256), lambda i: (0, 0))],
        out_specs=pl.BlockSpec((2, 5, bn, 128), lambda i: (0, 0, i, 0)),
        out_shape=jax.ShapeDtypeStruct((2, 5, n, 128), F32),
    )(h0, btabs, wmh)


def _update_body(lnext, h, agg, gate, wut, wm0, wm1, bu, btabs, wmhn, hout, tout):
    z = (h[...] @ wut[...] + agg[0] @ wm0[...] + agg[1] @ wm1[...] + bu[...])
    hn = jnp.maximum(z, 0.0) * gate[...]
    hout[...] = hn
    if lnext is not None:
        _emit_T(lnext, hn @ wmhn[...], btabs, tout)


def _update(l, h, agg, gate, btabs, p, n, bn):
    wu = p['g%d_wu' % l]
    args = [h, agg, gate, wu[0:256], wu[256:384], wu[384:512],
            p['g%d_bu' % l].reshape(1, 256), btabs]
    in_specs = [pl.BlockSpec((bn, 256), lambda i: (i, 0)),
                pl.BlockSpec((2, bn, 128), lambda i: (0, i, 0)),
                pl.BlockSpec((bn, 1), lambda i: (i, 0)),
                pl.BlockSpec((256, 256), lambda i: (0, 0)),
                pl.BlockSpec((128, 256), lambda i: (0, 0)),
                pl.BlockSpec((128, 256), lambda i: (0, 0)),
                pl.BlockSpec((1, 256), lambda i: (0, 0)),
                pl.BlockSpec((24, 256), lambda i: (0, 0))]
    out_shape = [jax.ShapeDtypeStruct((n, 256), F32)]
    out_specs = [pl.BlockSpec((bn, 256), lambda i: (i, 0))]
    lnext = l + 1 if l < 2 else None
    if lnext is not None:
        args.append(p['g%d_wm' % lnext][0:256])
        in_specs.append(pl.BlockSpec((256, 256), lambda i: (0, 0)))
        out_shape.append(jax.ShapeDtypeStruct((2, 5, n, 128), F32))
        out_specs.append(pl.BlockSpec((2, 5, bn, 128), lambda i: (0, 0, i, 0)))
        body = functools.partial(_update_body, lnext)
    else:
        body = lambda h_, a_, g_, w1, w2, w3, b_, bt_, ho: _update_body(
            None, h_, a_, g_, w1, w2, w3, b_, bt_, None, ho, None)
    return pl.pallas_call(
        body, grid=(n // bn,), in_specs=in_specs,
        out_specs=out_specs, out_shape=out_shape)(*args)


def _final_body(h3, degp, pvn, cw1, cb1, cw2, cb2, wa, wb, wv1, wv2, bb1,
                hf_o, viol_o, props_o, qa_o, qb_o):
    deg = degp[...]
    pv = pvn[...]
    viol = jnp.maximum(deg - pv, 0.0)
    hf = h3[...] / (1.0 + viol)
    hf_o[...] = hf
    viol_o[...] = viol
    t = jnp.maximum(hf @ cw1[...] + cb1[...], 0.0)
    props_o[...] = t @ cw2[...] + cb2[...]
    qa_o[...] = hf @ wa[...] + pv * wv1[...] + bb1[...]
    qb_o[...] = hf @ wb[...] + pv * wv2[...]


def _final(h3, degp, pvn, p, n, bn):
    w1 = p['btc_w1']
    args = [h3, degp, pvn, p['cp_w1'], p['cp_b1'].reshape(1, 128),
            p['cp_w2'], p['cp_b2'].reshape(1, 32),
            w1[0:256], w1[256:512], w1[512:513], w1[513:514],
            p['btc_b1'].reshape(1, 128)]
    in_specs = [pl.BlockSpec((bn, 256), lambda i: (i, 0)),
                pl.BlockSpec((bn, 1), lambda i: (i, 0)),
                pl.BlockSpec((bn, 1), lambda i: (i, 0)),
                pl.BlockSpec((256, 128), lambda i: (0, 0)),
                pl.BlockSpec((1, 128), lambda i: (0, 0)),
                pl.BlockSpec((128, 32), lambda i: (0, 0)),
                pl.BlockSpec((1, 32), lambda i: (0, 0)),
                pl.BlockSpec((256, 128), lambda i: (0, 0)),
                pl.BlockSpec((256, 128), lambda i: (0, 0)),
                pl.BlockSpec((1, 128), lambda i: (0, 0)),
                pl.BlockSpec((1, 128), lambda i: (0, 0)),
                pl.BlockSpec((1, 128), lambda i: (0, 0))]
    out_shape = [jax.ShapeDtypeStruct((n, 256), F32),
                 jax.ShapeDtypeStruct((n, 1), F32),
                 jax.ShapeDtypeStruct((n, 32), F32),
                 jax.ShapeDtypeStruct((n, 128), F32),
                 jax.ShapeDtypeStruct((n, 128), F32)]
    out_specs = [pl.BlockSpec((bn, 256), lambda i: (i, 0)),
                 pl.BlockSpec((bn, 1), lambda i: (i, 0)),
                 pl.BlockSpec((bn, 32), lambda i: (i, 0)),
                 pl.BlockSpec((bn, 128), lambda i: (i, 0)),
                 pl.BlockSpec((bn, 128), lambda i: (i, 0))]
    return pl.pallas_call(_final_body, grid=(n // bn,), in_specs=in_specs,
                          out_specs=out_specs, out_shape=out_shape)(*args)


def _bondmlp_body(b1, ow, w2, b2, w3, b3, out):
    be = b1.shape[0]
    t = jnp.maximum(b1[...] @ w2[...] + b2[...], 0.0)
    logits = t @ w3[...] + b3[...]
    o = ow[...]
    noble = ((o & 4) > 0).astype(F32)
    m2 = ((o & 2) > 0).astype(F32)
    m3 = ((o & 1) > 0).astype(F32)
    pen1 = -100.0 * noble - 50.0 * m3
    ci = lax.broadcasted_iota(I32, (be, 4), 1)
    out[...] = (logits + jnp.where(ci >= 1, pen1, 0.0)
                + jnp.where(ci == 2, -50.0 * m2, 0.0))


def _bondmlp(b1, ow, p, e, be):
    args = [b1, ow, p['btc_w2'], p['btc_b2'].reshape(1, 64),
            p['btc_w3'], p['btc_b3'].reshape(1, 4)]
    in_specs = [pl.BlockSpec((be, 128), lambda i: (i, 0)),
                pl.BlockSpec((be, 1), lambda i: (i, 0)),
                pl.BlockSpec((128, 64), lambda i: (0, 0)),
                pl.BlockSpec((1, 64), lambda i: (0, 0)),
                pl.BlockSpec((64, 4), lambda i: (0, 0)),
                pl.BlockSpec((1, 4), lambda i: (0, 0))]
    return pl.pallas_call(
        _bondmlp_body, grid=(e // be,), in_specs=in_specs,
        out_specs=pl.BlockSpec((be, 4), lambda i: (i, 0)),
        out_shape=jax.ShapeDtypeStruct((e, 4), F32))(*args)


def _zero_rows(zbuf, nrow, ncolv, value=0.0):
    def zrow(r, _):
        for k in range(ncolv):
            zbuf[r, pl.ds(16 * k, 16)] = jnp.full((16,), value, F32)
        return 0
    lax.fori_loop(0, nrow, zrow, 0)


def _edge_agg(tflat, g, col, n, npad, e):
    nchunk = e // CH
    per_tile = npad // 16
    mesh = plsc.VectorSubcoreMesh(core_axis_name="c", subcore_axis_name="s")

    @functools.partial(
        pl.kernel,
        out_type=jax.ShapeDtypeStruct((2, npad, 128), F32),
        mesh=mesh,
        scratch_types=[
            pltpu.VMEM((2, CH), I32),
            pltpu.VMEM((2, CH), I32),
            pltpu.VMEM((2, CH), I32),
            pltpu.VMEM((2, CH, 128), F32),
            pltpu.VMEM_SHARED((npad, 128), F32),
            pltpu.SemaphoreType.DMA,
            pltpu.SemaphoreType.DMA,
        ])
    def k(t_hbm, g_hbm, col_hbm, out_hbm, gbuf, g2buf, cbuf, msg,
          agg_sh, semA, semB):
        c = lax.axis_index("c")
        s = lax.axis_index("s")
        zbuf = msg.at[0]
        _zero_rows(zbuf, 128, 8)
        base = s * per_tile
        for q in range(per_tile // 128):
            pltpu.sync_copy(zbuf.at[pl.ds(0, 128)],
                            agg_sh.at[pl.ds(base + 128 * q, 128)])
        plsc.subcore_barrier()
        coff = c * (5 * n)
        sems = (semA, semB)

        def prep(i, b):
            j = s + 16 * i
            pltpu.sync_copy(g_hbm.at[pl.ds(j * CH, CH)], gbuf.at[b])
            for kk in range(8):
                sl = pl.ds(16 * kk, 16)
                g2buf[b, sl] = gbuf[b, sl] + coff
            pltpu.sync_copy(col_hbm.at[pl.ds(j * CH, CH)], cbuf.at[b])

        def start(b):
            pltpu.async_copy(t_hbm.at[g2buf.at[b]], msg.at[b], sems[b])

        def wait(b):
            pltpu.make_async_copy(t_hbm.at[g2buf.at[b]], msg.at[b],
                                  sems[b]).wait()

        def scat(b):
            pltpu.sync_copy(msg.at[b], agg_sh.at[cbuf.at[b]], add=True)

        def v(i):
            return s + 16 * i < nchunk

        prep(0, 0)
        start(0)

        def body(ii, _):
            i0 = 2 * ii
            i1 = i0 + 1
            i2 = i0 + 2

            @pl.when(v(i1))
            def _():
                prep(i1, 1)
                start(1)

            @pl.when(v(i0))
            def _():
                wait(0)
                scat(0)

            @pl.when(v(i2))
            def _():
                prep(i2, 0)
                start(0)

            @pl.when(v(i1))
            def _():
                wait(1)
                scat(1)
            return 0

        lax.fori_loop(0, (nchunk // 16 + 2) // 2, body, 0)
        plsc.subcore_barrier()
        pltpu.sync_copy(agg_sh.at[pl.ds(base, per_tile)],
                        out_hbm.at[c, pl.ds(base, per_tile)])

    return k(tflat, g, col)


def _degree_tc_body(col, out):
    i = pl.program_id(0)
    cb = col[...]
    be = cb.shape[0]
    a = cb // 128
    b = cb % 128
    oha = (a == lax.broadcasted_iota(I32, (be, 80), 1)).astype(jnp.bfloat16)
    ohb = (b == lax.broadcasted_iota(I32, (be, 128), 1)).astype(jnp.bfloat16)
    acc = lax.dot_general(oha, ohb, (((0,), (0,)), ((), ())),
                          preferred_element_type=F32)

    @pl.when(i == 0)
    def _():
        out[...] = acc

    @pl.when(i > 0)
    def _():
        out[...] += acc


def _degree(col1, e, be):
    return pl.pallas_call(
        _degree_tc_body,
        grid=(e // be,),
        in_specs=[pl.BlockSpec((be, 1), lambda i: (i, 0))],
        out_specs=pl.BlockSpec((80, 128), lambda i: (0, 0)),
        out_shape=jax.ShapeDtypeStruct((80, 128), F32),
    )(col1)


def _bond_edge(qa, qb, row, col, n, e):
    nchunk = e // CH
    mesh = plsc.VectorSubcoreMesh(core_axis_name="c", subcore_axis_name="s")

    @functools.partial(
        pl.kernel,
        out_type=jax.ShapeDtypeStruct((e, 128), F32),
        mesh=mesh,
        scratch_types=[
            pltpu.VMEM((2, CH), I32),
            pltpu.VMEM((2, CH), I32),
            pltpu.VMEM((2, CH, 128), F32),
            pltpu.VMEM((2, CH, 128), F32),
            pltpu.SemaphoreType.DMA,
            pltpu.SemaphoreType.DMA,
            pltpu.SemaphoreType.DMA,
            pltpu.SemaphoreType.DMA,
            pltpu.SemaphoreType.DMA,
            pltpu.SemaphoreType.DMA,
        ])
    def k(qa_hbm, qb_hbm, row_hbm, col_hbm, b1_o,
          rbuf, cbuf, bufA, bufB, semA0, semA1, semB0, semB1, semW0, semW1):
        c = lax.axis_index("c")
        s = lax.axis_index("s")
        w = s * 2 + c
        semsA = (semA0, semA1)
        semsB = (semB0, semB1)
        semsW = (semW0, semW1)

        def prep(i, b):
            j = w + 32 * i
            pltpu.sync_copy(row_hbm.at[pl.ds(j * CH, CH)], rbuf.at[b])
            pltpu.sync_copy(col_hbm.at[pl.ds(j * CH, CH)], cbuf.at[b])

        def start(b):
            pltpu.async_copy(qa_hbm.at[rbuf.at[b]], bufA.at[b], semsA[b])
            pltpu.async_copy(qb_hbm.at[cbuf.at[b]], bufB.at[b], semsB[b])

        def wait(b):
            pltpu.make_async_copy(qa_hbm.at[rbuf.at[b]], bufA.at[b],
                                  semsA[b]).wait()
            pltpu.make_async_copy(qb_hbm.at[cbuf.at[b]], bufB.at[b],
                                  semsB[b]).wait()

        def compute(i, b):
            j = w + 32 * i

            def rbody(r, _):
                for kk in range(8):
                    sl = pl.ds(16 * kk, 16)
                    bufA[b, r, sl] = jnp.maximum(
                        bufA[b, r, sl] + bufB[b, r, sl], 0.0)
                return 0

            lax.fori_loop(0, CH, rbody, 0)
            pltpu.async_copy(bufA.at[b], b1_o.at[pl.ds(j * CH, CH)], semsW[b])

        def wait_w(b):
            pltpu.make_async_copy(bufA.at[b], b1_o.at[pl.ds(0, CH)],
                                  semsW[b]).wait()

        def v(i):
            return w + 32 * i < nchunk

        prep(0, 0)
        start(0)

        def body(ii, _):
            i0 = 2 * ii
            i1 = i0 + 1
            i2 = i0 + 2

            @pl.when(v(i1))
            def _():
                prep(i1, 1)

                @pl.when(ii > 0)
                def _():
                    wait_w(1)
                start(1)

            @pl.when(v(i0))
            def _():
                wait(0)
                compute(i0, 0)

            @pl.when(v(i2))
            def _():
                prep(i2, 0)
                wait_w(0)
                start(0)

            @pl.when(v(i1))
            def _():
                wait(1)
                compute(i1, 1)
            return 0

        lax.fori_loop(0, (nchunk // 32 + 2) // 2, body, 0)
        wait_w(0)
        wait_w(1)

    return k(qa, qb, row, col)


def _owflags(wfl, row, col, n, e):
    nchunk = e // CH
    mesh = plsc.VectorSubcoreMesh(core_axis_name="c", subcore_axis_name="s")

    @functools.partial(
        pl.kernel,
        out_type=jax.ShapeDtypeStruct((e,), I32),
        mesh=mesh,
        scratch_types=[
            pltpu.VMEM((CH,), I32),
            pltpu.VMEM((CH,), I32),
            pltpu.VMEM((CH,), I32),
            pltpu.VMEM((n,), I32),
        ],
        compiler_params=pltpu.CompilerParams(needs_layout_passes=False))
    def k(w_hbm, row_hbm, col_hbm, ow_o, rbuf, cbuf, owb, wtile):
        c = lax.axis_index("c")
        s = lax.axis_index("s")
        w = s * 2 + c
        pltpu.sync_copy(w_hbm, wtile)

        def body(i, _):
            j = w + 32 * i

            @pl.when(j < nchunk)
            def _():
                pltpu.sync_copy(row_hbm.at[pl.ds(j * CH, CH)], rbuf)
                pltpu.sync_copy(col_hbm.at[pl.ds(j * CH, CH)], cbuf)
                for k in range(8):
                    sl = pl.ds(16 * k, 16)
                    wr = plsc.load_gather(wtile, [rbuf[sl]])
                    wc = plsc.load_gather(wtile, [cbuf[sl]])
                    owb[sl] = wr | wc
                pltpu.sync_copy(owb, ow_o.at[pl.ds(j * CH, CH)])
            return 0

        lax.fori_loop(0, nchunk // 32 + 1, body, 0)

    return k(wfl, row, col)


def kernel(x, edge_index, edge_attr, batch, params):
    p = params
    n, e = x.shape[0], edge_index.shape[1]
    bn = 1000
    row = edge_index[0].astype(I32)
    col = edge_index[1].astype(I32)

    bigtab, btabs = _tables(p)
    nodef = _nodef(x, bigtab, n, bn)
    h0 = nodef[:, 0:256]
    vlog = nodef[:, 256:264]
    pvn = nodef[:, 264:265]
    gate = nodef[:, 265:266]
    wfl = nodef[:, 266].astype(I32)

    gr = _edge_prep(edge_attr[:, 0].reshape(e // CH, CH),
                    row.reshape(e // CH, CH), n)
    g = gr.reshape(e)

    npad = ((n + 639) // 640) * 640
    degp = _degree(col.reshape(e, 1), e, 16000).reshape(80 * 128, 1)

    t = _prep_T0(h0, btabs, p['g0_wm'][0:256], n, bn)
    h = h0
    for l in range(3):
        agg = _edge_agg(t.reshape(10 * n, 128), g, col, n, npad, e)
        outs = _update(l, h, agg, gate, btabs, p, n, bn)
        if l < 2:
            h, t = outs
        else:
            h = outs[0]

    hf, viol, props, qa, qb = _final(h, degp, pvn, p, n, bn)
    b1 = _bond_edge(qa, qb, row, col, n, e)
    ow = _owflags(wfl, row, col, n, e)
    blog = _bondmlp(b1, ow.reshape(e, 1), p, e, 8000)
    return (hf, props, vlog, blog, viol.reshape(n))


# speedup vs baseline: 9.9604x; 1.0264x over previous
import functools

import jax
import jax.numpy as jnp
from jax import lax
from jax.experimental import pallas as pl
from jax.experimental.pallas import tpu as pltpu
from jax.experimental.pallas import tpu_sc as plsc

F32 = jnp.float32
I32 = jnp.int32
CH = 128


def _tables_body(atab, bond, vw1, vb1, vw2, vb2,
                 wmb0, bm0, wmb1, bm1, wmb2, bm2, bigtab, btabs):
    a = atab[...]
    h1 = jnp.maximum(a @ vw1[...] + vb1[...], 0.0)
    vl = h1 @ vw2[...] + vb2[...]
    mx = jnp.max(vl, axis=1, keepdims=True)
    io8 = lax.broadcasted_iota(I32, (16, 8), 1).astype(F32)
    idx = jnp.min(jnp.where(vl == mx, io8, 8.0), axis=1, keepdims=True)
    pv = idx + 1.0
    oh = (io8 == idx).astype(F32)
    gate = jnp.clip(pv, 1.0, 8.0) / 8.0
    rowid = lax.broadcasted_iota(I32, (16, 1), 0).astype(F32)
    noble = jnp.logical_or(rowid == 4.0, rowid == 5.0).astype(F32)
    wflag = 4.0 * noble + 2.0 * (pv <= 2.0).astype(F32) + (pv <= 1.0).astype(F32)
    z184 = jnp.zeros((16, 184), F32)
    z117 = jnp.zeros((16, 117), F32)
    bigtab[...] = jnp.concatenate([a, oh, z184, vl, pv, gate, wflag, z117], axis=1)
    b = bond[...]
    t0 = b @ wmb0[...] + bm0[...]
    t1 = b @ wmb1[...] + bm1[...]
    t2 = b @ wmb2[...] + bm2[...]
    btabs[...] = jnp.concatenate([t0, t1, t2], axis=0)


def _tables(p):
    atab = jnp.pad(p['atom_table'], ((0, 5), (0, 0)))
    bond = jnp.pad(p['bond_table'], ((0, 3), (0, 0)))
    args = [atab, bond, p['vp_w1'], p['vp_b1'].reshape(1, 32),
            p['vp_w2'], p['vp_b2'].reshape(1, 8)]
    for l in range(3):
        args += [p['g%d_wm' % l][256:320], p['g%d_bm' % l].reshape(1, 256)]
    return pl.pallas_call(
        _tables_body,
        out_shape=[jax.ShapeDtypeStruct((16, 384), F32),
                   jax.ShapeDtypeStruct((24, 256), F32)],
    )(*args)


def _nodef_body(x, bigtab, nodef):
    b = x.shape[0]
    at = jnp.clip(x[:, 0:1].astype(I32), 0, 10)
    oh = (at == lax.broadcasted_iota(I32, (b, 16), 1)).astype(F32)
    nodef[...] = oh @ bigtab[...]


def _nodef(x, bigtab, n, bn):
    return pl.pallas_call(
        _nodef_body,
        grid=(n // bn,),
        in_specs=[pl.BlockSpec((bn, x.shape[1]), lambda i: (i, 0)),
                  pl.BlockSpec((16, 384), lambda i: (0, 0))],
        out_specs=pl.BlockSpec((bn, 384), lambda i: (i, 0)),
        out_shape=jax.ShapeDtypeStruct((n, 384), F32),
    )(x, bigtab)


def _edge_prep_body(n, ea0, rowr, gr):
    bt = jnp.clip(ea0[...].astype(I32), 0, 4)
    gr[...] = bt * n + rowr[...]


def _edge_prep(ea0r, rowr, n):
    r, c = ea0r.shape
    return pl.pallas_call(
        functools.partial(_edge_prep_body, n),
        out_shape=jax.ShapeDtypeStruct((r, c), I32),
    )(ea0r, rowr)


def _emit_T(l, P, btabs_ref, T_ref):
    bt = btabs_ref[...]
    for c in range(2):
        for b in range(5):
            T_ref[c, b] = jnp.maximum(
                P[:, c * 128:(c + 1) * 128]
                + bt[8 * l + b:8 * l + b + 1, c * 128:(c + 1) * 128], 0.0)


def _prep_T0_body(h, btabs, wmh, T):
    _emit_T(0, h[...] @ wmh[...], btabs, T)


def _prep_T0(h0, btabs, wmh, n, bn):
    return pl.pallas_call(
        _prep_T0_body,
        grid=(n // bn,),
        in_specs=[pl.BlockSpec((bn, 256), lambda i: (i, 0)),
                  pl.BlockSpec((24, 256), lambda i: (0, 0)),
                  pl.BlockSpec((256, 256), lambda i: (0, 0))],
        out_specs=pl.BlockSpec((2, 5, bn, 128), lambda i: (0, 0, i, 0)),
        out_shape=jax.ShapeDtypeStruct((2, 5, n, 128), F32),
    )(h0, btabs, wmh)


def _update_body(lnext, h, agg, gate, wut, wm0, wm1, bu, btabs, wmhn, hout, tout):
    z = (h[...] @ wut[...] + agg[0] @ wm0[...] + agg[1] @ wm1[...] + bu[...])
    hn = jnp.maximum(z, 0.0) * gate[...]
    hout[...] = hn
    if lnext is not None:
        _emit_T(lnext, hn @ wmhn[...], btabs, tout)


def _update(l, h, agg, gate, btabs, p, n, bn):
    wu = p['g%d_wu' % l]
    args = [h, agg, gate, wu[0:256], wu[256:384], wu[384:512],
            p['g%d_bu' % l].reshape(1, 256), btabs]
    in_specs = [pl.BlockSpec((bn, 256), lambda i: (i, 0)),
                pl.BlockSpec((2, bn, 128), lambda i: (0, i, 0)),
                pl.BlockSpec((bn, 1), lambda i: (i, 0)),
                pl.BlockSpec((256, 256), lambda i: (0, 0)),
                pl.BlockSpec((128, 256), lambda i: (0, 0)),
                pl.BlockSpec((128, 256), lambda i: (0, 0)),
                pl.BlockSpec((1, 256), lambda i: (0, 0)),
                pl.BlockSpec((24, 256), lambda i: (0, 0))]
    out_shape = [jax.ShapeDtypeStruct((n, 256), F32)]
    out_specs = [pl.BlockSpec((bn, 256), lambda i: (i, 0))]
    lnext = l + 1 if l < 2 else None
    if lnext is not None:
        args.append(p['g%d_wm' % lnext][0:256])
        in_specs.append(pl.BlockSpec((256, 256), lambda i: (0, 0)))
        out_shape.append(jax.ShapeDtypeStruct((2, 5, n, 128), F32))
        out_specs.append(pl.BlockSpec((2, 5, bn, 128), lambda i: (0, 0, i, 0)))
        body = functools.partial(_update_body, lnext)
    else:
        body = lambda h_, a_, g_, w1, w2, w3, b_, bt_, ho: _update_body(
            None, h_, a_, g_, w1, w2, w3, b_, bt_, None, ho, None)
    return pl.pallas_call(
        body, grid=(n // bn,), in_specs=in_specs,
        out_specs=out_specs, out_shape=out_shape)(*args)


def _final_body(h3, degp, pvn, cw1, cb1, cw2, cb2, wa, wb, wv1, wv2, bb1,
                hf_o, viol_o, props_o, qa_o, qb_o):
    deg = degp[...]
    pv = pvn[...]
    viol = jnp.maximum(deg - pv, 0.0)
    hf = h3[...] / (1.0 + viol)
    hf_o[...] = hf
    viol_o[...] = viol
    t = jnp.maximum(hf @ cw1[...] + cb1[...], 0.0)
    props_o[...] = t @ cw2[...] + cb2[...]
    qa_o[...] = hf @ wa[...] + pv * wv1[...] + bb1[...]
    qb_o[...] = hf @ wb[...] + pv * wv2[...]


def _final(h3, degp, pvn, p, n, bn):
    w1 = p['btc_w1']
    args = [h3, degp, pvn, p['cp_w1'], p['cp_b1'].reshape(1, 128),
            p['cp_w2'], p['cp_b2'].reshape(1, 32),
            w1[0:256], w1[256:512], w1[512:513], w1[513:514],
            p['btc_b1'].reshape(1, 128)]
    in_specs = [pl.BlockSpec((bn, 256), lambda i: (i, 0)),
                pl.BlockSpec((bn, 1), lambda i: (i, 0)),
                pl.BlockSpec((bn, 1), lambda i: (i, 0)),
                pl.BlockSpec((256, 128), lambda i: (0, 0)),
                pl.BlockSpec((1, 128), lambda i: (0, 0)),
                pl.BlockSpec((128, 32), lambda i: (0, 0)),
                pl.BlockSpec((1, 32), lambda i: (0, 0)),
                pl.BlockSpec((256, 128), lambda i: (0, 0)),
                pl.BlockSpec((256, 128), lambda i: (0, 0)),
                pl.BlockSpec((1, 128), lambda i: (0, 0)),
                pl.BlockSpec((1, 128), lambda i: (0, 0)),
                pl.BlockSpec((1, 128), lambda i: (0, 0))]
    out_shape = [jax.ShapeDtypeStruct((n, 256), F32),
                 jax.ShapeDtypeStruct((n, 1), F32),
                 jax.ShapeDtypeStruct((n, 32), F32),
                 jax.ShapeDtypeStruct((n, 128), F32),
                 jax.ShapeDtypeStruct((n, 128), F32)]
    out_specs = [pl.BlockSpec((bn, 256), lambda i: (i, 0)),
                 pl.BlockSpec((bn, 1), lambda i: (i, 0)),
                 pl.BlockSpec((bn, 32), lambda i: (i, 0)),
                 pl.BlockSpec((bn, 128), lambda i: (i, 0)),
                 pl.BlockSpec((bn, 128), lambda i: (i, 0))]
    return pl.pallas_call(_final_body, grid=(n // bn,), in_specs=in_specs,
                          out_specs=out_specs, out_shape=out_shape)(*args)


def _bondmlp_body(b1, ow, w2, b2, w3, b3, out):
    be = b1.shape[0]
    t = jnp.maximum(b1[...] @ w2[...] + b2[...], 0.0)
    logits = t @ w3[...] + b3[...]
    o = ow[...]
    noble = ((o & 4) > 0).astype(F32)
    m2 = ((o & 2) > 0).astype(F32)
    m3 = ((o & 1) > 0).astype(F32)
    pen1 = -100.0 * noble - 50.0 * m3
    ci = lax.broadcasted_iota(I32, (be, 4), 1)
    out[...] = (logits + jnp.where(ci >= 1, pen1, 0.0)
                + jnp.where(ci == 2, -50.0 * m2, 0.0))


def _bondmlp(b1, ow, p, e, be):
    args = [b1, ow, p['btc_w2'], p['btc_b2'].reshape(1, 64),
            p['btc_w3'], p['btc_b3'].reshape(1, 4)]
    in_specs = [pl.BlockSpec((be, 128), lambda i: (i, 0)),
                pl.BlockSpec((be, 1), lambda i: (i, 0)),
                pl.BlockSpec((128, 64), lambda i: (0, 0)),
                pl.BlockSpec((1, 64), lambda i: (0, 0)),
                pl.BlockSpec((64, 4), lambda i: (0, 0)),
                pl.BlockSpec((1, 4), lambda i: (0, 0))]
    return pl.pallas_call(
        _bondmlp_body, grid=(e // be,), in_specs=in_specs,
        out_specs=pl.BlockSpec((be, 4), lambda i: (i, 0)),
        out_shape=jax.ShapeDtypeStruct((e, 4), F32))(*args)


def _zero_rows(zbuf, nrow, ncolv, value=0.0):
    def zrow(r, _):
        for k in range(ncolv):
            zbuf[r, pl.ds(16 * k, 16)] = jnp.full((16,), value, F32)
        return 0
    lax.fori_loop(0, nrow, zrow, 0)


def _edge_agg(tflat, g, col, n, npad, e):
    nchunk = e // CH
    per_tile = npad // 16
    mesh = plsc.VectorSubcoreMesh(core_axis_name="c", subcore_axis_name="s")

    @functools.partial(
        pl.kernel,
        out_type=jax.ShapeDtypeStruct((2, npad, 128), F32),
        mesh=mesh,
        scratch_types=[
            pltpu.VMEM((2, CH), I32),
            pltpu.VMEM((2, CH), I32),
            pltpu.VMEM((2, CH), I32),
            pltpu.VMEM((2, CH, 128), F32),
            pltpu.VMEM_SHARED((npad, 128), F32),
            pltpu.SemaphoreType.DMA,
            pltpu.SemaphoreType.DMA,
        ])
    def k(t_hbm, g_hbm, col_hbm, out_hbm, gbuf, g2buf, cbuf, msg,
          agg_sh, semA, semB):
        c = lax.axis_index("c")
        s = lax.axis_index("s")
        zbuf = msg.at[0]
        _zero_rows(zbuf, 128, 8)
        base = s * per_tile
        for q in range(per_tile // 128):
            pltpu.sync_copy(zbuf.at[pl.ds(0, 128)],
                            agg_sh.at[pl.ds(base + 128 * q, 128)])
        plsc.subcore_barrier()
        coff = c * (5 * n)
        sems = (semA, semB)

        def prep(i, b):
            j = s + 16 * i
            pltpu.sync_copy(g_hbm.at[pl.ds(j * CH, CH)], gbuf.at[b])
            for kk in range(8):
                sl = pl.ds(16 * kk, 16)
                g2buf[b, sl] = gbuf[b, sl] + coff
            pltpu.sync_copy(col_hbm.at[pl.ds(j * CH, CH)], cbuf.at[b])

        def start(b):
            pltpu.async_copy(t_hbm.at[g2buf.at[b]], msg.at[b], sems[b])

        def wait(b):
            pltpu.make_async_copy(t_hbm.at[g2buf.at[b]], msg.at[b],
                                  sems[b]).wait()

        def scat(b):
            pltpu.sync_copy(msg.at[b], agg_sh.at[cbuf.at[b]], add=True)

        def v(i):
            return s + 16 * i < nchunk

        prep(0, 0)
        start(0)

        def body(ii, _):
            i0 = 2 * ii
            i1 = i0 + 1
            i2 = i0 + 2

            @pl.when(v(i1))
            def _():
                prep(i1, 1)
                start(1)

            @pl.when(v(i0))
            def _():
                wait(0)
                scat(0)

            @pl.when(v(i2))
            def _():
                prep(i2, 0)
                start(0)

            @pl.when(v(i1))
            def _():
                wait(1)
                scat(1)
            return 0

        lax.fori_loop(0, (nchunk // 16 + 2) // 2, body, 0)
        plsc.subcore_barrier()
        pltpu.sync_copy(agg_sh.at[pl.ds(base, per_tile)],
                        out_hbm.at[c, pl.ds(base, per_tile)])

    return k(tflat, g, col)


def _degree_tc_body(col, out):
    i = pl.program_id(0)
    cb = col[...]
    be = cb.shape[0]
    a = cb // 128
    b = cb % 128
    oha = (a == lax.broadcasted_iota(I32, (be, 80), 1)).astype(jnp.bfloat16)
    ohb = (b == lax.broadcasted_iota(I32, (be, 128), 1)).astype(jnp.bfloat16)
    acc = lax.dot_general(oha, ohb, (((0,), (0,)), ((), ())),
                          preferred_element_type=F32)

    @pl.when(i == 0)
    def _():
        out[...] = acc

    @pl.when(i > 0)
    def _():
        out[...] += acc


def _degree(col1, e, be):
    return pl.pallas_call(
        _degree_tc_body,
        grid=(e // be,),
        in_specs=[pl.BlockSpec((be, 1), lambda i: (i, 0))],
        out_specs=pl.BlockSpec((80, 128), lambda i: (0, 0)),
        out_shape=jax.ShapeDtypeStruct((80, 128), F32),
    )(col1)


def _bond_edge(qa, qb, row, col, n, e):
    nchunk = e // CH
    mesh = plsc.VectorSubcoreMesh(core_axis_name="c", subcore_axis_name="s")

    @functools.partial(
        pl.kernel,
        out_type=jax.ShapeDtypeStruct((e, 128), F32),
        mesh=mesh,
        scratch_types=[
            pltpu.VMEM((2, CH), I32),
            pltpu.VMEM((2, CH), I32),
            pltpu.VMEM((2, CH, 128), F32),
            pltpu.VMEM((2, CH, 128), F32),
            pltpu.SemaphoreType.DMA,
            pltpu.SemaphoreType.DMA,
            pltpu.SemaphoreType.DMA,
            pltpu.SemaphoreType.DMA,
            pltpu.SemaphoreType.DMA,
            pltpu.SemaphoreType.DMA,
        ])
    def k(qa_hbm, qb_hbm, row_hbm, col_hbm, b1_o,
          rbuf, cbuf, bufA, bufB, semA0, semA1, semB0, semB1, semW0, semW1):
        c = lax.axis_index("c")
        s = lax.axis_index("s")
        w = s * 2 + c
        semsA = (semA0, semA1)
        semsB = (semB0, semB1)
        semsW = (semW0, semW1)

        def prep(i, b):
            j = w + 32 * i
            pltpu.sync_copy(row_hbm.at[pl.ds(j * CH, CH)], rbuf.at[b])
            pltpu.sync_copy(col_hbm.at[pl.ds(j * CH, CH)], cbuf.at[b])

        def start(b):
            pltpu.async_copy(qa_hbm.at[rbuf.at[b]], bufA.at[b], semsA[b])
            pltpu.async_copy(qb_hbm.at[cbuf.at[b]], bufB.at[b], semsB[b])

        def wait(b):
            pltpu.make_async_copy(qa_hbm.at[rbuf.at[b]], bufA.at[b],
                                  semsA[b]).wait()
            pltpu.make_async_copy(qb_hbm.at[cbuf.at[b]], bufB.at[b],
                                  semsB[b]).wait()

        def compute(i, b):
            j = w + 32 * i

            def rbody(r, _):
                for kk in range(8):
                    sl = pl.ds(16 * kk, 16)
                    bufA[b, r, sl] = jnp.maximum(
                        bufA[b, r, sl] + bufB[b, r, sl], 0.0)
                return 0

            lax.fori_loop(0, CH, rbody, 0)
            pltpu.async_copy(bufA.at[b], b1_o.at[pl.ds(j * CH, CH)], semsW[b])

        def wait_w(b):
            pltpu.make_async_copy(bufA.at[b], b1_o.at[pl.ds(0, CH)],
                                  semsW[b]).wait()

        def v(i):
            return w + 32 * i < nchunk

        prep(0, 0)
        start(0)

        def body(ii, _):
            i0 = 2 * ii
            i1 = i0 + 1
            i2 = i0 + 2

            @pl.when(v(i1))
            def _():
                prep(i1, 1)

                @pl.when(ii > 0)
                def _():
                    wait_w(1)
                start(1)

            @pl.when(v(i0))
            def _():
                wait(0)
                compute(i0, 0)

            @pl.when(v(i2))
            def _():
                prep(i2, 0)
                wait_w(0)
                start(0)

            @pl.when(v(i1))
            def _():
                wait(1)
                compute(i1, 1)
            return 0

        lax.fori_loop(0, (nchunk // 32 + 2) // 2, body, 0)
        wait_w(0)
        wait_w(1)

    return k(qa, qb, row, col)


def _owflags(wfl, row, col, n, e):
    chw = 640
    nchunk = e // chw
    mesh = plsc.VectorSubcoreMesh(core_axis_name="c", subcore_axis_name="s")

    @functools.partial(
        pl.kernel,
        out_type=jax.ShapeDtypeStruct((e,), I32),
        mesh=mesh,
        scratch_types=[
            pltpu.VMEM((chw,), I32),
            pltpu.VMEM((chw,), I32),
            pltpu.VMEM((chw,), I32),
            pltpu.VMEM((n,), I32),
        ],
        compiler_params=pltpu.CompilerParams(needs_layout_passes=False))
    def k(w_hbm, row_hbm, col_hbm, ow_o, rbuf, cbuf, owb, wtile):
        c = lax.axis_index("c")
        s = lax.axis_index("s")
        w = s * 2 + c
        pltpu.sync_copy(w_hbm, wtile)

        def body(i, _):
            j = w + 32 * i

            @pl.when(j < nchunk)
            def _():
                pltpu.sync_copy(row_hbm.at[pl.ds(j * chw, chw)], rbuf)
                pltpu.sync_copy(col_hbm.at[pl.ds(j * chw, chw)], cbuf)

                def gbody(q, _):
                    sl = pl.ds(q * 16, 16)
                    wr = plsc.load_gather(wtile, [rbuf[sl]])
                    wc = plsc.load_gather(wtile, [cbuf[sl]])
                    owb[sl] = wr | wc
                    return 0

                lax.fori_loop(0, chw // 16, gbody, 0)
                pltpu.sync_copy(owb, ow_o.at[pl.ds(j * chw, chw)])
            return 0

        lax.fori_loop(0, nchunk // 32 + 1, body, 0)

    return k(wfl, row, col)


def kernel(x, edge_index, edge_attr, batch, params):
    p = params
    n, e = x.shape[0], edge_index.shape[1]
    bn = 1000
    row = edge_index[0].astype(I32)
    col = edge_index[1].astype(I32)

    bigtab, btabs = _tables(p)
    nodef = _nodef(x, bigtab, n, bn)
    h0 = nodef[:, 0:256]
    vlog = nodef[:, 256:264]
    pvn = nodef[:, 264:265]
    gate = nodef[:, 265:266]
    wfl = nodef[:, 266].astype(I32)

    gr = _edge_prep(edge_attr[:, 0].reshape(e // CH, CH),
                    row.reshape(e // CH, CH), n)
    g = gr.reshape(e)

    npad = ((n + 639) // 640) * 640
    degp = _degree(col.reshape(e, 1), e, 16000).reshape(80 * 128, 1)

    t = _prep_T0(h0, btabs, p['g0_wm'][0:256], n, bn)
    h = h0
    for l in range(3):
        agg = _edge_agg(t.reshape(10 * n, 128), g, col, n, npad, e)
        outs = _update(l, h, agg, gate, btabs, p, n, bn)
        if l < 2:
            h, t = outs
        else:
            h = outs[0]

    hf, viol, props, qa, qb = _final(h, degp, pvn, p, n, bn)
    b1 = _bond_edge(qa, qb, row, col, n, e)
    ow = _owflags(wfl, row, col, n, e)
    blog = _bondmlp(b1, ow.reshape(e, 1), p, e, 8000)
    return (hf, props, vlog, blog, viol.reshape(n))
